# baseline (device time: 98554 ns/iter reference)
import jax
import jax.numpy as jnp
from jax import lax
from jax.experimental import pallas as pl
from jax.experimental.pallas import tpu as pltpu

N_DEV = 4
SQ = 512
SQC = SQ // N_DEV
SKV_SHARD = 2048
HQ = 8
DH = 128
DM = 1024
SCALE = 0.08838834764831843
KV_CHUNK = 1024


def kernel(x, Wq, Wo, K_ext, V_ext):
    x2 = x.reshape(SQ, DM)
    K = K_ext.reshape(SKV_SHARD, HQ, DH)
    V = V_ext.reshape(SKV_SHARD, HQ, DH)

    def body(x_ref, wq_ref, wo_ref, k_ref, v_ref, out_ref,
             acc_ref, m_ref, l_ref, rs_acc, rs_ml, ml_stage,
             rs_acc_send, rs_acc_recv, rs_ml_send, rs_ml_recv,
             ag_send, ag_recv):
        my = lax.axis_index("i")
        left = lax.rem(my + N_DEV - 1, N_DEV)
        right = lax.rem(my + 1, N_DEV)
        diag = lax.rem(my + 2, N_DEV)

        barrier = pltpu.get_barrier_semaphore()
        for nbr in (left, right):
            pl.semaphore_signal(barrier, inc=1, device_id=(nbr,),
                                device_id_type=pl.DeviceIdType.MESH)
        pl.semaphore_wait(barrier, 2)

        acc_ref[:, :] = jnp.zeros((SQ, DM), jnp.float32)

        def compute_chunk(c):
            rows = pl.ds(c * SQC, SQC)
            qc = jnp.dot(x_ref[rows, :], wq_ref[:, :],
                         preferred_element_type=jnp.float32)
            for h in range(HQ):
                cols = slice(DH * h, DH * (h + 1))
                qh = qc[:, cols]

                def kv_body(j, carry, h=h, cols=cols, qh=qh, rows=rows):
                    m_prev, l_prev = carry
                    krows = pl.ds(j * KV_CHUNK, KV_CHUNK)
                    kh = k_ref[krows, h, :]
                    vh = v_ref[krows, h, :]
                    s = lax.dot_general(qh, kh, (((1,), (1,)), ((), ())),
                                        preferred_element_type=jnp.float32)
                    s = s * SCALE
                    mj = jnp.max(s, axis=1, keepdims=True)
                    m_new = jnp.maximum(m_prev, mj)
                    p = jnp.exp(s - m_new)
                    alpha = jnp.exp(m_prev - m_new)
                    l_new = (l_prev * alpha
                             + jnp.sum(p, axis=1, keepdims=True))
                    acc_ref[rows, cols] = (
                        acc_ref[rows, cols] * alpha
                        + jnp.dot(p, vh, preferred_element_type=jnp.float32))
                    return (m_new, l_new)

                m_fin, l_fin = lax.fori_loop(
                    0, SKV_SHARD // KV_CHUNK, kv_body,
                    (jnp.full((SQC, 1), -jnp.inf, jnp.float32),
                     jnp.zeros((SQC, 1), jnp.float32)))
                m_ref[rows, h:h + 1] = m_fin
                l_ref[rows, h:h + 1] = l_fin

        erow = lax.broadcasted_iota(jnp.int32, (HQ, DM), 0)
        ecol = lax.broadcasted_iota(jnp.int32, (HQ, DM), 1)
        E = jnp.where(ecol // DH == erow, 1.0, 0.0).astype(jnp.float32)

        compute_chunk(my)
        for t in range(N_DEV - 1):
            cs = lax.rem(my - t + N_DEV, N_DEV)
            cn = lax.rem(my - t - 1 + N_DEV, N_DEV)
            srows = pl.ds(cs * SQC, SQC)
            ml_stage[t, :, 0:HQ] = m_ref[srows, :]
            ml_stage[t, :, HQ:2 * HQ] = l_ref[srows, :]
            acc_rdma = pltpu.make_async_remote_copy(
                src_ref=acc_ref.at[srows, :], dst_ref=rs_acc.at[t],
                send_sem=rs_acc_send.at[t], recv_sem=rs_acc_recv.at[t],
                device_id=(right,), device_id_type=pl.DeviceIdType.MESH)
            ml_rdma = pltpu.make_async_remote_copy(
                src_ref=ml_stage.at[t], dst_ref=rs_ml.at[t],
                send_sem=rs_ml_send.at[t], recv_sem=rs_ml_recv.at[t],
                device_id=(right,), device_id_type=pl.DeviceIdType.MESH)
            acc_rdma.start()
            ml_rdma.start()

            compute_chunk(cn)

            acc_rdma.wait()
            ml_rdma.wait()

            mrows = pl.ds(cn * SQC, SQC)
            m_old = m_ref[mrows, :]
            l_old = l_ref[mrows, :]
            m_r = rs_ml[t, :, 0:HQ]
            l_r = rs_ml[t, :, HQ:2 * HQ]
            m_new = jnp.maximum(m_old, m_r)
            ea = jnp.exp(m_old - m_new)
            eb = jnp.exp(m_r - m_new)
            m_ref[mrows, :] = m_new
            l_ref[mrows, :] = l_old * ea + l_r * eb
            ea_x = jnp.dot(ea, E, preferred_element_type=jnp.float32)
            eb_x = jnp.dot(eb, E, preferred_element_type=jnp.float32)
            acc_ref[mrows, :] = (acc_ref[mrows, :] * ea_x
                                 + rs_acc[t, :, :] * eb_x)

        own = lax.rem(my + 1, N_DEV)
        orows = pl.ds(own * SQC, SQC)
        linv = jnp.dot(1.0 / l_ref[orows, :], E,
                       preferred_element_type=jnp.float32)
        out_ref[orows, :] = jnp.dot(acc_ref[orows, :] * linv, wo_ref[:, :],
                                    preferred_element_type=jnp.float32)

        ag = []
        for k, tgt in enumerate((right, left, diag)):
            r = pltpu.make_async_remote_copy(
                src_ref=out_ref.at[orows, :], dst_ref=out_ref.at[orows, :],
                send_sem=ag_send.at[k], recv_sem=ag_recv.at[k],
                device_id=(tgt,), device_id_type=pl.DeviceIdType.MESH)
            r.start()
            ag.append(r)
        for r in ag:
            r.wait_send()
        for r in ag:
            r.wait_recv()

    out = pl.pallas_call(
        body,
        out_shape=jax.ShapeDtypeStruct((SQ, DM), jnp.float32),
        in_specs=[pl.BlockSpec(memory_space=pltpu.VMEM)] * 5,
        out_specs=pl.BlockSpec(memory_space=pltpu.VMEM),
        scratch_shapes=[
            pltpu.VMEM((SQ, DM), jnp.float32),
            pltpu.VMEM((SQ, HQ), jnp.float32),
            pltpu.VMEM((SQ, HQ), jnp.float32),
            pltpu.VMEM((N_DEV - 1, SQC, DM), jnp.float32),
            pltpu.VMEM((N_DEV - 1, SQC, 2 * HQ), jnp.float32),
            pltpu.VMEM((N_DEV - 1, SQC, 2 * HQ), jnp.float32),
            pltpu.SemaphoreType.DMA((N_DEV - 1,)),
            pltpu.SemaphoreType.DMA((N_DEV - 1,)),
            pltpu.SemaphoreType.DMA((N_DEV - 1,)),
            pltpu.SemaphoreType.DMA((N_DEV - 1,)),
            pltpu.SemaphoreType.DMA((3,)),
            pltpu.SemaphoreType.DMA((3,)),
        ],
        compiler_params=pltpu.CompilerParams(
            collective_id=0, vmem_limit_bytes=60 * 1024 * 1024),
    )(x2, Wq, Wo, K, V)
    return out.reshape(1, SQ, DM)
